# Initial kernel scaffold; baseline (speedup 1.0000x reference)
import jax
import jax.numpy as jnp
from jax.experimental import pallas as pl


def kernel(x, weights, control_points):
    raise NotImplementedError("write your pallas kernel here")


# speedup vs baseline: 819.4639x; 819.4639x over previous
import functools

import jax
import jax.numpy as jnp
from jax import lax
from jax.experimental import pallas as pl
from jax.experimental.pallas import tpu as pltpu
from jax.experimental.pallas import tpu_sc as plsc

_N = 256
_B = 8192
_L = 16
_NC, _NS = 2, 16
_NW = _NC * _NS
_BPW = _B // _NW
_NP = _N + 1
_TAB = _NP * _NP
_TABP = ((_TAB + 15) // 16) * 16
_INV_H = (_N - 1) / 2.0


def _tec_body(xs_hbm, ys_hbm, wtab_hbm, out_hbm, wtab_v, xq_v, yq_v, out_v, sem):
    c = lax.axis_index("c")
    s = lax.axis_index("s")
    wid = s * _NC + c
    base = wid * _BPW

    table_cp = pltpu.async_copy(wtab_hbm, wtab_v, sem)
    pltpu.sync_copy(xs_hbm.at[pl.ds(base, _BPW)], xq_v)
    pltpu.sync_copy(ys_hbm.at[pl.ds(base, _BPW)], yq_v)
    table_cp.wait()

    for chunk in range(_BPW // _L):
        off = chunk * _L
        xs = xq_v[pl.ds(off, _L)]
        ys = yq_v[pl.ds(off, _L)]

        jf = (xs + 1.0) * _INV_H
        yf = (ys + 1.0) * _INV_H
        j0 = jf.astype(jnp.int32)
        i0 = yf.astype(jnp.int32)
        u = jf - j0.astype(jnp.float32)
        v = yf - i0.astype(jnp.float32)

        def cubic_weights(t):
            t2 = t * t
            t3 = t2 * t
            w0 = -0.5 * (t3 - 2.0 * t2 + t)
            w1 = 1.5 * t3 - 2.5 * t2 + 1.0
            w2 = -1.5 * t3 + 2.0 * t2 + 0.5 * t
            w3 = 0.5 * (t3 - t2)
            return (w0, w1, w2, w3)

        wx = cubic_weights(u)
        wy = cubic_weights(v)

        idx00 = i0 * _NP + j0 - (_NP + 1)
        acc = jnp.zeros((_L,), jnp.float32)
        for di in range(4):
            for dj in range(4):
                g = plsc.load_gather(wtab_v, [idx00 + (di * _NP + dj)])
                acc = acc + g * (wy[di] * wx[dj])
        out_v[pl.ds(off, _L)] = acc

    pltpu.sync_copy(out_v, out_hbm.at[pl.ds(base, _BPW)])


@functools.partial(jax.jit, static_argnames=())
def _interp(xs, ys, wtab):
    run = pl.kernel(
        _tec_body,
        out_type=jax.ShapeDtypeStruct((_B,), jnp.float32),
        mesh=plsc.VectorSubcoreMesh(core_axis_name="c", subcore_axis_name="s"),
        compiler_params=pltpu.CompilerParams(needs_layout_passes=False),
        scratch_types=[
            pltpu.VMEM((_TABP,), jnp.float32),
            pltpu.VMEM((_BPW,), jnp.float32),
            pltpu.VMEM((_BPW,), jnp.float32),
            pltpu.VMEM((_BPW,), jnp.float32),
            pltpu.SemaphoreType.DMA,
        ],
    )
    return run(xs, ys, wtab)


def kernel(x, weights, control_points):
    wt = jnp.pad(weights.reshape(_N, _N), ((0, 1), (0, 1)))
    wflat = jnp.pad(wt.reshape(-1), (0, _TABP - _TAB))
    out = _interp(x[:, 0], x[:, 1], wflat)
    return (out, x)


# speedup vs baseline: 945.4403x; 1.1537x over previous
import functools

import jax
import jax.numpy as jnp
from jax import lax
from jax.experimental import pallas as pl
from jax.experimental.pallas import tpu as pltpu
from jax.experimental.pallas import tpu_sc as plsc

_N = 256
_B = 8192
_L = 16
_NC, _NS = 2, 16
_NW = _NC * _NS
_BPW = _B // _NW
_NP = _N + 1
_TAB = _NP * _NP
_TABP = ((_TAB + 15) // 16) * 16
_INV_H = (_N - 1) / 2.0


def _tec_body(xs_hbm, ys_hbm, wtab_hbm, out_hbm, wtab_sp, wtab_v, xq_v, yq_v,
              out_v, sem):
    c = lax.axis_index("c")
    s = lax.axis_index("s")
    wid = s * _NC + c
    base = wid * _BPW

    @pl.when(s == 0)
    def _():
        pltpu.sync_copy(wtab_hbm, wtab_sp)
    plsc.subcore_barrier()
    table_cp = pltpu.async_copy(wtab_sp, wtab_v, sem)
    pltpu.sync_copy(xs_hbm.at[pl.ds(base, _BPW)], xq_v)
    pltpu.sync_copy(ys_hbm.at[pl.ds(base, _BPW)], yq_v)
    table_cp.wait()

    for chunk in range(_BPW // _L):
        off = chunk * _L
        xs = xq_v[pl.ds(off, _L)]
        ys = yq_v[pl.ds(off, _L)]

        jf = (xs + 1.0) * _INV_H
        yf = (ys + 1.0) * _INV_H
        j0 = jf.astype(jnp.int32)
        i0 = yf.astype(jnp.int32)
        u = jf - j0.astype(jnp.float32)
        v = yf - i0.astype(jnp.float32)

        def cubic_weights(t):
            t2 = t * t
            t3 = t2 * t
            w0 = -0.5 * (t3 - 2.0 * t2 + t)
            w1 = 1.5 * t3 - 2.5 * t2 + 1.0
            w2 = -1.5 * t3 + 2.0 * t2 + 0.5 * t
            w3 = 0.5 * (t3 - t2)
            return (w0, w1, w2, w3)

        wx = cubic_weights(u)
        wy = cubic_weights(v)

        idx00 = i0 * _NP + j0 - (_NP + 1)
        acc = jnp.zeros((_L,), jnp.float32)
        for di in range(4):
            for dj in range(4):
                g = plsc.load_gather(wtab_v, [idx00 + (di * _NP + dj)])
                acc = acc + g * (wy[di] * wx[dj])
        out_v[pl.ds(off, _L)] = acc

    pltpu.sync_copy(out_v, out_hbm.at[pl.ds(base, _BPW)])


@functools.partial(jax.jit, static_argnames=())
def _interp(xs, ys, wtab):
    run = pl.kernel(
        _tec_body,
        out_type=jax.ShapeDtypeStruct((_B,), jnp.float32),
        mesh=plsc.VectorSubcoreMesh(core_axis_name="c", subcore_axis_name="s"),
        compiler_params=pltpu.CompilerParams(needs_layout_passes=False),
        scratch_types=[
            pltpu.VMEM_SHARED((_TABP,), jnp.float32),
            pltpu.VMEM((_TABP,), jnp.float32),
            pltpu.VMEM((_BPW,), jnp.float32),
            pltpu.VMEM((_BPW,), jnp.float32),
            pltpu.VMEM((_BPW,), jnp.float32),
            pltpu.SemaphoreType.DMA,
        ],
    )
    return run(xs, ys, wtab)


def kernel(x, weights, control_points):
    wt = jnp.pad(weights.reshape(_N, _N), ((0, 1), (0, 1)))
    wflat = jnp.pad(wt.reshape(-1), (0, _TABP - _TAB))
    out = _interp(x[:, 0], x[:, 1], wflat)
    return (out, x)
